# baseline (speedup 1.0000x reference)
import functools

import jax
import jax.numpy as jnp
from jax import lax
from jax.experimental import pallas as pl
from jax.experimental.pallas import tpu as pltpu
from jax.experimental.pallas import tpu_sc as plsc

N_NODES = 10000
N_EDGES = 320000
D = 128
NC = 2
NS = 16
NW = NC * NS
E_PER_TILE = N_EDGES // NW
CHUNK = 40
N_CHUNKS = E_PER_TILE // CHUNK
NB = 8
RCHUNK = 40
N_RCHUNK = N_NODES // RCHUNK
RC_PER_TILE = -(-N_RCHUNK // NS)


def _segsum_body(ea_hbm, dst_hbm, out_hbm, acc, *scr):
    idxs = scr[0:NB]
    rows = scr[NB:2 * NB]
    sem_l = scr[2 * NB:3 * NB]
    sem_s = scr[3 * NB:4 * NB]
    zbuf = rows[NB - 1]
    cid = lax.axis_index("c")
    sid = lax.axis_index("s")
    wid = sid * NC + cid

    base = wid * E_PER_TILE

    def load_start(i, b):
        off = pl.multiple_of(base + i * CHUNK, 8)
        pltpu.async_copy(dst_hbm.at[pl.ds(off, CHUNK)], idxs[b], sem_l[b])
        pltpu.async_copy(ea_hbm.at[pl.ds(off, CHUNK)], rows[b], sem_l[b])

    def load_wait(i, b):
        off = pl.multiple_of(base + i * CHUNK, 8)
        pltpu.make_async_copy(dst_hbm.at[pl.ds(off, CHUNK)], idxs[b], sem_l[b]).wait()
        pltpu.make_async_copy(ea_hbm.at[pl.ds(off, CHUNK)], rows[b], sem_l[b]).wait()

    def scat_start(b):
        pltpu.async_copy(rows[b], acc.at[idxs[b]], sem_s[b], add=True)

    def scat_wait(b):
        pltpu.make_async_copy(rows[b], acc.at[idxs[b]], sem_s[b]).wait()

    for b in range(NB - 1):
        load_start(b, b)

    def zb(i, carry):
        r = i // (D // 16)
        j = i % (D // 16)
        zbuf[r, pl.ds(j * 16, 16)] = jnp.zeros((16,), jnp.float32)
        return carry

    lax.fori_loop(0, RCHUNK * (D // 16), zb, None)

    for k in range(RC_PER_TILE):
        c = k * NS + sid

        @pl.when(c < N_RCHUNK)
        def _(k=k, c=c):
            pltpu.async_copy(
                zbuf, acc.at[pl.ds(c * RCHUNK, RCHUNK)], sem_s[k % NB]
            )

    for k in range(RC_PER_TILE):
        c = k * NS + sid

        @pl.when(c < N_RCHUNK)
        def _(k=k, c=c):
            pltpu.make_async_copy(
                zbuf, acc.at[pl.ds(c * RCHUNK, RCHUNK)], sem_s[k % NB]
            ).wait()

    plsc.subcore_barrier()

    load_wait(0, 0)
    scat_start(0)
    load_start(NB - 1, NB - 1)
    for b in range(1, NB):
        load_wait(b, b)
        scat_wait(b - 1)
        scat_start(b)
        load_start(b + NB - 1, (b - 1) % NB)

    def step(j, carry):
        for b in range(NB):
            i = NB * j + b
            load_wait(i, b)
            scat_wait((b - 1) % NB)
            scat_start(b)

            @pl.when(i + NB - 1 < N_CHUNKS)
            def _(i=i, b=b):
                load_start(i + NB - 1, (b - 1) % NB)

        return carry

    lax.fori_loop(1, N_CHUNKS // NB, step, None)

    last = (N_CHUNKS // NB) * NB
    for b in range(N_CHUNKS - last):
        i = last + b
        load_wait(i, b)
        scat_wait((b - 1) % NB)
        scat_start(b)
    scat_wait((N_CHUNKS - 1) % NB)

    plsc.subcore_barrier()

    for k in range(RC_PER_TILE):
        c = k * NS + sid

        @pl.when(c < N_RCHUNK)
        def _(k=k, c=c):
            pltpu.async_copy(
                acc.at[pl.ds(c * RCHUNK, RCHUNK)],
                out_hbm.at[cid, pl.ds(c * RCHUNK, RCHUNK)],
                sem_l[k % NB],
            )

    for k in range(RC_PER_TILE):
        c = k * NS + sid

        @pl.when(c < N_RCHUNK)
        def _(k=k, c=c):
            pltpu.make_async_copy(
                acc.at[pl.ds(c * RCHUNK, RCHUNK)],
                out_hbm.at[cid, pl.ds(c * RCHUNK, RCHUNK)],
                sem_l[k % NB],
            ).wait()


def _segsum_sc(edge_attr, dst):
    mesh = plsc.VectorSubcoreMesh(
        core_axis_name="c", subcore_axis_name="s", num_cores=NC, num_subcores=NS
    )
    f = pl.kernel(
        _segsum_body,
        out_type=jax.ShapeDtypeStruct((NC, N_NODES, D), jnp.float32),
        mesh=mesh,
        scratch_types=(
            [pltpu.VMEM_SHARED((N_NODES, D), jnp.float32)]
            + [pltpu.VMEM((CHUNK,), jnp.int32) for _ in range(NB)]
            + [pltpu.VMEM((CHUNK, D), jnp.float32) for _ in range(NB)]
            + [pltpu.SemaphoreType.DMA for _ in range(2 * NB)]
        ),
    )
    return f(edge_attr, dst)


def _update_body(p_ref, x_ref, w_ref, b_ref, o_ref):
    agg = p_ref[0] + p_ref[1]
    o_ref[...] = (
        jnp.dot(agg, w_ref[:D], preferred_element_type=jnp.float32)
        + jnp.dot(x_ref[...], w_ref[D:], preferred_element_type=jnp.float32)
        + b_ref[...]
    )


def _update_tc(partials, x, W, b):
    RB = 2000
    return pl.pallas_call(
        _update_body,
        grid=(N_NODES // RB,),
        in_specs=[
            pl.BlockSpec((2, RB, D), lambda i: (0, i, 0)),
            pl.BlockSpec((RB, D), lambda i: (i, 0)),
            pl.BlockSpec((2 * D, D), lambda i: (0, 0)),
            pl.BlockSpec((1, D), lambda i: (0, 0)),
        ],
        out_specs=pl.BlockSpec((RB, D), lambda i: (i, 0)),
        out_shape=jax.ShapeDtypeStruct((N_NODES, D), jnp.float32),
    )(partials, x, W, b.reshape(1, D))


@jax.jit
def kernel(x, edge_attr, edge_index, W, b):
    dst = edge_index[1]
    partials = _segsum_sc(edge_attr, dst)
    return _update_tc(partials, x, W, b)


# speedup vs baseline: 1.0886x; 1.0886x over previous
import functools

import jax
import jax.numpy as jnp
from jax import lax
from jax.experimental import pallas as pl
from jax.experimental.pallas import tpu as pltpu
from jax.experimental.pallas import tpu_sc as plsc

N_NODES = 10000
N_EDGES = 320000
D = 128
NC = 2
NS = 16
NW = NC * NS
E_PER_TILE = N_EDGES // NW
CHUNK = 40
N_CHUNKS = E_PER_TILE // CHUNK
NB = 8
RCHUNK = 40
N_RCHUNK = N_NODES // RCHUNK
RC_PER_TILE = -(-N_RCHUNK // NS)


def _segsum_body(ea_hbm, dst_hbm, out_hbm, acc, *scr):
    idxs = scr[0:NB]
    rows = scr[NB:2 * NB]
    sem_l = scr[2 * NB:3 * NB]
    sem_s = scr[3 * NB:4 * NB]
    zbuf = rows[NB - 1]
    cid = lax.axis_index("c")
    sid = lax.axis_index("s")
    wid = sid * NC + cid

    base = wid * E_PER_TILE

    def load_start(i, b):
        off = pl.multiple_of(base + i * CHUNK, 8)
        pltpu.async_copy(dst_hbm.at[pl.ds(N_EDGES + off, CHUNK)], idxs[b], sem_l[b])
        pltpu.async_copy(ea_hbm.at[pl.ds(off, CHUNK)], rows[b], sem_l[b])

    def load_wait(i, b):
        off = pl.multiple_of(base + i * CHUNK, 8)
        pltpu.make_async_copy(
            dst_hbm.at[pl.ds(N_EDGES + off, CHUNK)], idxs[b], sem_l[b]
        ).wait()
        pltpu.make_async_copy(ea_hbm.at[pl.ds(off, CHUNK)], rows[b], sem_l[b]).wait()

    def scat_start(b):
        pltpu.async_copy(rows[b], acc.at[idxs[b]], sem_s[b], add=True)

    def scat_wait(b):
        pltpu.make_async_copy(rows[b], acc.at[idxs[b]], sem_s[b]).wait()

    for b in range(NB - 1):
        load_start(b, b)

    def zb(i, carry):
        r = i // (D // 16)
        j = i % (D // 16)
        zbuf[r, pl.ds(j * 16, 16)] = jnp.zeros((16,), jnp.float32)
        return carry

    lax.fori_loop(0, RCHUNK * (D // 16), zb, None)

    for k in range(RC_PER_TILE):
        c = k * NS + sid

        @pl.when(c < N_RCHUNK)
        def _(k=k, c=c):
            pltpu.async_copy(
                zbuf, acc.at[pl.ds(c * RCHUNK, RCHUNK)], sem_s[k % NB]
            )

    for k in range(RC_PER_TILE):
        c = k * NS + sid

        @pl.when(c < N_RCHUNK)
        def _(k=k, c=c):
            pltpu.make_async_copy(
                zbuf, acc.at[pl.ds(c * RCHUNK, RCHUNK)], sem_s[k % NB]
            ).wait()

    plsc.subcore_barrier()

    load_wait(0, 0)
    scat_start(0)
    load_start(NB - 1, NB - 1)
    for b in range(1, NB):
        load_wait(b, b)
        scat_wait(b - 1)
        scat_start(b)
        load_start(b + NB - 1, (b - 1) % NB)

    def step(j, carry):
        for b in range(NB):
            i = NB * j + b
            load_wait(i, b)
            scat_wait((b - 1) % NB)
            scat_start(b)

            @pl.when(i + NB - 1 < N_CHUNKS)
            def _(i=i, b=b):
                load_start(i + NB - 1, (b - 1) % NB)

        return carry

    lax.fori_loop(1, N_CHUNKS // NB, step, None)

    last = (N_CHUNKS // NB) * NB
    for b in range(N_CHUNKS - last):
        i = last + b
        load_wait(i, b)
        scat_wait((b - 1) % NB)
        scat_start(b)
    scat_wait((N_CHUNKS - 1) % NB)

    plsc.subcore_barrier()

    for k in range(RC_PER_TILE):
        c = k * NS + sid

        @pl.when(c < N_RCHUNK)
        def _(k=k, c=c):
            pltpu.async_copy(
                acc.at[pl.ds(c * RCHUNK, RCHUNK)],
                out_hbm.at[cid, pl.ds(c * RCHUNK, RCHUNK)],
                sem_l[k % NB],
            )

    for k in range(RC_PER_TILE):
        c = k * NS + sid

        @pl.when(c < N_RCHUNK)
        def _(k=k, c=c):
            pltpu.make_async_copy(
                acc.at[pl.ds(c * RCHUNK, RCHUNK)],
                out_hbm.at[cid, pl.ds(c * RCHUNK, RCHUNK)],
                sem_l[k % NB],
            ).wait()


def _segsum_sc(edge_attr, edge_index_flat):
    mesh = plsc.VectorSubcoreMesh(
        core_axis_name="c", subcore_axis_name="s", num_cores=NC, num_subcores=NS
    )
    f = pl.kernel(
        _segsum_body,
        out_type=jax.ShapeDtypeStruct((NC, N_NODES, D), jnp.float32),
        mesh=mesh,
        scratch_types=(
            [pltpu.VMEM_SHARED((N_NODES, D), jnp.float32)]
            + [pltpu.VMEM((CHUNK,), jnp.int32) for _ in range(NB)]
            + [pltpu.VMEM((CHUNK, D), jnp.float32) for _ in range(NB)]
            + [pltpu.SemaphoreType.DMA for _ in range(2 * NB)]
        ),
    )
    return f(edge_attr, edge_index_flat)


def _update_body(p_ref, x_ref, w_ref, b_ref, o_ref):
    agg = p_ref[0] + p_ref[1]
    o_ref[...] = (
        jnp.dot(agg, w_ref[:D], preferred_element_type=jnp.float32)
        + jnp.dot(x_ref[...], w_ref[D:], preferred_element_type=jnp.float32)
        + b_ref[...]
    )


def _update_tc(partials, x, W, b):
    RB = 2000
    return pl.pallas_call(
        _update_body,
        grid=(N_NODES // RB,),
        in_specs=[
            pl.BlockSpec((2, RB, D), lambda i: (0, i, 0)),
            pl.BlockSpec((RB, D), lambda i: (i, 0)),
            pl.BlockSpec((2 * D, D), lambda i: (0, 0)),
            pl.BlockSpec((1, D), lambda i: (0, 0)),
        ],
        out_specs=pl.BlockSpec((RB, D), lambda i: (i, 0)),
        out_shape=jax.ShapeDtypeStruct((N_NODES, D), jnp.float32),
    )(partials, x, W, b.reshape(1, D))


@jax.jit
def kernel(x, edge_attr, edge_index, W, b):
    partials = _segsum_sc(edge_attr, edge_index.reshape(2 * N_EDGES))
    return _update_tc(partials, x, W, b)
